# baseline (speedup 1.0000x reference)
import jax
import jax.numpy as jnp
from jax import lax
from jax.experimental import pallas as pl
from jax.experimental.pallas import tpu as pltpu
from jax.experimental.pallas import tpu_sc as plsc

VOCAB = 1000000
DIM = 64
B = 16384
L = 50

N = B * L
NC, NS = 2, 16
NW = NC * NS
N_PER_W = N // NW
N_CHUNKS = N_PER_W // (2 * L)
PH = 1
CPP = N_CHUNKS // PH
K = 2
D = 2 * K
LP = 56
R_TOT = B * LP
RPW = R_TOT // NW


def _emb_body(idx_hbm, table_hbm, out_hbm, idx_v, rows_v, *sems):
    gsem = sems[:D]
    osem = sems[D:]
    cid = lax.axis_index("c")
    sid = lax.axis_index("s")
    wid = sid * NC + cid
    rbase = wid * RPW

    def gather_fire(mloc, b):
        pltpu.async_copy(table_hbm.at[idx_v.at[mloc]], rows_v.at[b], gsem[b])

    def gather_wait(b):
        pltpu.make_async_copy(
            table_hbm.at[pl.ds(0, 128)], rows_v.at[b], gsem[b]
        ).wait()

    def out_fire(ph, mloc, b):
        ro = rbase + (ph * CPP + mloc) * (2 * LP)
        pltpu.async_copy(
            rows_v.at[b, pl.ds(0, 2 * LP)], out_hbm.at[pl.ds(ro, 2 * LP)], osem[b]
        )

    def out_wait(b):
        pltpu.make_async_copy(
            rows_v.at[b, pl.ds(0, 2 * LP)], out_hbm.at[pl.ds(0, 2 * LP)], osem[b]
        ).wait()

    for ph in range(PH):
        pltpu.sync_copy(idx_hbm.at[wid, pl.ds(ph * CPP, CPP)], idx_v)

        for b in range(K):
            gather_fire(b, b)
        for j in range(K):
            gather_fire(j + K, j + K)
            gather_wait(j % D)
            out_fire(ph, j, j % D)

        def group(g, carry):
            j0 = K + g * D
            for i in range(D):
                b = (K + i) % D
                bf = (K + i + K) % D
                j = j0 + i
                out_wait(bf)
                gather_fire(j + K, bf)
                gather_wait(b)
                out_fire(ph, j, b)
            return carry

        lax.fori_loop(0, (CPP - 2 * K) // D, group, 0)

        for i in range(K):
            j = CPP - K + i
            b = j % D
            out_wait((j + K) % D)
            gather_wait(b)
            out_fire(ph, j, b)
        for i in range(K):
            j = CPP - K + i
            out_wait(j % D)


@jax.jit
def _emb_lookup(idx, emb_weight):
    idx4 = jnp.pad(
        idx.reshape(NW, N_CHUNKS, 2, L).astype(jnp.int32),
        ((0, 0), (0, 0), (0, 0), (0, LP - L)),
    )
    idxg = jnp.pad(
        idx4.reshape(NW, N_CHUNKS, 2 * LP), ((0, 0), (0, 0), (0, 128 - 2 * LP))
    )
    wp = jnp.pad(emb_weight, ((0, 0), (0, 128 - DIM)))
    mesh = plsc.VectorSubcoreMesh(core_axis_name="c", subcore_axis_name="s")
    run = pl.kernel(
        _emb_body,
        out_type=jax.ShapeDtypeStruct((R_TOT, 128), jnp.float32),
        mesh=mesh,
        scratch_types=[
            pltpu.VMEM((CPP, 128), jnp.int32),
            pltpu.VMEM((D, 128, 128), jnp.float32),
        ] + [pltpu.SemaphoreType.DMA] * (2 * D),
        compiler_params=pltpu.CompilerParams(use_tc_tiling_on_sc=False),
    )
    op = run(idxg, wp)
    return op.reshape(B, LP, 128)[:, :L, :DIM]


def kernel(idx, emb_weight):
    return _emb_lookup(idx, emb_weight)


# speedup vs baseline: 9.8446x; 9.8446x over previous
import jax
import jax.numpy as jnp
from jax import lax
from jax.experimental import pallas as pl
from jax.experimental.pallas import tpu as pltpu
from jax.experimental.pallas import tpu_sc as plsc

VOCAB = 1000000
DIM = 64
B = 16384
L = 50

N = B * L
NC, NS = 2, 16
NW = NC * NS
N_PER_W = N // NW
N_CHUNKS = N_PER_W // (2 * L)
PH = 2
CPP = N_CHUNKS // PH
K = 4
D = 2 * K
LP = 56
R_TOT = B * LP
RPW = R_TOT // NW


def _emb_body(idx_hbm, table_hbm, out_hbm, idx_v, rows_v, *sems):
    gsem = sems[:D]
    osem = sems[D:]
    cid = lax.axis_index("c")
    sid = lax.axis_index("s")
    wid = sid * NC + cid
    rbase = wid * RPW

    def gather_fire(mloc, b):
        pltpu.async_copy(table_hbm.at[idx_v.at[mloc]], rows_v.at[b], gsem[b])

    def gather_wait(b):
        pltpu.make_async_copy(
            table_hbm.at[pl.ds(0, 2 * LP)], rows_v.at[b], gsem[b]
        ).wait()

    def out_fire(ph, mloc, b):
        ro = rbase + (ph * CPP + mloc) * (2 * LP)
        pltpu.async_copy(
            rows_v.at[b, pl.ds(0, 2 * LP)], out_hbm.at[pl.ds(ro, 2 * LP)], osem[b]
        )

    def out_wait(b):
        pltpu.make_async_copy(
            rows_v.at[b, pl.ds(0, 2 * LP)], out_hbm.at[pl.ds(0, 2 * LP)], osem[b]
        ).wait()

    for ph in range(PH):
        pltpu.sync_copy(idx_hbm.at[wid, pl.ds(ph * CPP, CPP)], idx_v)

        for b in range(K):
            gather_fire(b, b)
        for j in range(K):
            gather_fire(j + K, j + K)
            gather_wait(j % D)
            out_fire(ph, j, j % D)

        def group(g, carry):
            j0 = K + g * D
            for i in range(D):
                b = (K + i) % D
                bf = (K + i + K) % D
                j = j0 + i
                out_wait(bf)
                gather_fire(j + K, bf)
                gather_wait(b)
                out_fire(ph, j, b)
            return carry

        lax.fori_loop(0, (CPP - 2 * K) // D, group, 0)

        for i in range(K):
            j = CPP - K + i
            b = j % D
            out_wait((j + K) % D)
            gather_wait(b)
            out_fire(ph, j, b)
        for i in range(K):
            j = CPP - K + i
            out_wait(j % D)


@jax.jit
def _emb_lookup(idx, emb_weight):
    idx4 = jnp.pad(
        idx.reshape(NW, N_CHUNKS, 2, L).astype(jnp.int32),
        ((0, 0), (0, 0), (0, 0), (0, LP - L)),
        mode="wrap",
    )
    idxg = idx4.reshape(NW, N_CHUNKS, 2 * LP)
    wp = jnp.pad(emb_weight, ((0, 0), (0, 128 - DIM)))
    mesh = plsc.VectorSubcoreMesh(core_axis_name="c", subcore_axis_name="s")
    run = pl.kernel(
        _emb_body,
        out_type=jax.ShapeDtypeStruct((R_TOT, 128), jnp.float32),
        mesh=mesh,
        scratch_types=[
            pltpu.VMEM((CPP, 2 * LP), jnp.int32),
            pltpu.VMEM((D, 2 * LP, 128), jnp.float32),
        ] + [pltpu.SemaphoreType.DMA] * (2 * D),
        compiler_params=pltpu.CompilerParams(use_tc_tiling_on_sc=False),
    )
    op = run(idxg, wp)
    return op.reshape(B, LP, 128)[:, :L, :DIM]


def kernel(idx, emb_weight):
    return _emb_lookup(idx, emb_weight)


# speedup vs baseline: 10.6858x; 1.0855x over previous
import jax
import jax.numpy as jnp
from jax import lax
from jax.experimental import pallas as pl
from jax.experimental.pallas import tpu as pltpu
from jax.experimental.pallas import tpu_sc as plsc

VOCAB = 1000000
DIM = 64
B = 16384
L = 50

N = B * L
NC, NS = 2, 16
NW = NC * NS
N_PER_W = N // NW
N_CHUNKS = N_PER_W // (2 * L)
PH = 2
CPP = N_CHUNKS // PH
K = 4
D = 2 * K
LP = 56
R_TOT = B * LP
RPW = R_TOT // NW


def _emb_body(idx_hbm, table_hbm, out_hbm, idx_v, rows_v, *sems):
    gsem = sems[:D]
    osem = sems[D:]
    cid = lax.axis_index("c")
    sid = lax.axis_index("s")
    wid = sid * NC + cid
    rbase = wid * RPW

    def gather_fire(mloc, b):
        pltpu.async_copy(table_hbm.at[idx_v.at[mloc]], rows_v.at[b], gsem[b])

    def gather_wait(b):
        pltpu.make_async_copy(
            table_hbm.at[pl.ds(0, 2 * LP)], rows_v.at[b], gsem[b]
        ).wait()

    def out_fire(ph, mloc, b):
        ro = rbase + (ph * CPP + mloc) * (2 * LP)
        pltpu.async_copy(
            rows_v.at[b, pl.ds(0, 2 * LP), pl.ds(0, DIM)],
            out_hbm.at[pl.ds(ro, 2 * LP), pl.ds(0, DIM)],
            osem[b],
        )

    def out_wait(b):
        pltpu.make_async_copy(
            rows_v.at[b, pl.ds(0, 2 * LP), pl.ds(0, DIM)],
            out_hbm.at[pl.ds(0, 2 * LP), pl.ds(0, DIM)],
            osem[b],
        ).wait()

    for ph in range(PH):
        pltpu.sync_copy(idx_hbm.at[wid, pl.ds(ph * CPP, CPP)], idx_v)

        for b in range(K):
            gather_fire(b, b)
        for j in range(K):
            gather_fire(j + K, j + K)
            gather_wait(j % D)
            out_fire(ph, j, j % D)

        def group(g, carry):
            j0 = K + g * D
            for i in range(D):
                b = (K + i) % D
                bf = (K + i + K) % D
                j = j0 + i
                out_wait(bf)
                gather_fire(j + K, bf)
                gather_wait(b)
                out_fire(ph, j, b)
            return carry

        lax.fori_loop(0, (CPP - 2 * K) // D, group, 0)

        for i in range(K):
            j = CPP - K + i
            b = j % D
            out_wait((j + K) % D)
            gather_wait(b)
            out_fire(ph, j, b)
        for i in range(K):
            j = CPP - K + i
            out_wait(j % D)


@jax.jit
def _emb_lookup(idx, emb_weight):
    idx4 = jnp.pad(
        idx.reshape(NW, N_CHUNKS, 2, L).astype(jnp.int32),
        ((0, 0), (0, 0), (0, 0), (0, LP - L)),
        mode="wrap",
    )
    idxg = idx4.reshape(NW, N_CHUNKS, 2 * LP)
    wp = jnp.pad(emb_weight, ((0, 0), (0, 128 - DIM)))
    mesh = plsc.VectorSubcoreMesh(core_axis_name="c", subcore_axis_name="s")
    run = pl.kernel(
        _emb_body,
        out_type=jax.ShapeDtypeStruct((R_TOT, 128), jnp.float32),
        mesh=mesh,
        scratch_types=[
            pltpu.VMEM((CPP, 2 * LP), jnp.int32),
            pltpu.VMEM((D, 2 * LP, 128), jnp.float32),
        ] + [pltpu.SemaphoreType.DMA] * (2 * D),
        compiler_params=pltpu.CompilerParams(use_tc_tiling_on_sc=False),
    )
    op = run(idxg, wp)
    return op.reshape(B, LP, 128)[:, :L, :DIM]


def kernel(idx, emb_weight):
    return _emb_lookup(idx, emb_weight)


# speedup vs baseline: 11.0035x; 1.0297x over previous
import jax
import jax.numpy as jnp
from jax import lax
from jax.experimental import pallas as pl
from jax.experimental.pallas import tpu as pltpu
from jax.experimental.pallas import tpu_sc as plsc

VOCAB = 1000000
DIM = 64
B = 16384
L = 50

N = B * L
NC, NS = 2, 16
NW = NC * NS
N_PER_W = N // NW
CH = 2 * L
N_CHUNKS = N_PER_W // CH
PH = 2
CPP = N_CHUNKS // PH
K = 4
D = 2 * K
LP = 56
R_TOT = B * LP
RPW = R_TOT // NW


def _emb_body(idx_hbm, table_hbm, out_hbm, idx_v, rows_v, *sems):
    gsem = sems[:D]
    osem = sems[D:]
    cid = lax.axis_index("c")
    sid = lax.axis_index("s")
    wid = sid * NC + cid
    rbase = wid * RPW

    def gather_fire(mloc, b):
        pltpu.async_copy(table_hbm.at[idx_v.at[mloc]], rows_v.at[b], gsem[b])

    def gather_wait(b):
        pltpu.make_async_copy(
            table_hbm.at[pl.ds(0, CH)], rows_v.at[b], gsem[b]
        ).wait()

    def out_fire(ph, mloc, b):
        ro = rbase + (ph * CPP + mloc) * (2 * LP)
        pltpu.async_copy(
            rows_v.at[b, pl.ds(0, L), pl.ds(0, DIM)],
            out_hbm.at[pl.ds(ro, L), pl.ds(0, DIM)],
            osem[b],
        )
        pltpu.async_copy(
            rows_v.at[b, pl.ds(L, L), pl.ds(0, DIM)],
            out_hbm.at[pl.ds(ro + LP, L), pl.ds(0, DIM)],
            osem[b],
        )

    def out_wait(b):
        for _ in range(2):
            pltpu.make_async_copy(
                rows_v.at[b, pl.ds(0, L), pl.ds(0, DIM)],
                out_hbm.at[pl.ds(0, L), pl.ds(0, DIM)],
                osem[b],
            ).wait()

    for ph in range(PH):
        pltpu.sync_copy(idx_hbm.at[wid, pl.ds(ph * CPP, CPP)], idx_v)

        for b in range(K):
            gather_fire(b, b)
        for j in range(K):
            gather_fire(j + K, j + K)
            gather_wait(j % D)
            out_fire(ph, j, j % D)

        def group(g, carry):
            j0 = K + g * D
            for i in range(D):
                b = (K + i) % D
                bf = (K + i + K) % D
                j = j0 + i
                out_wait(bf)
                gather_fire(j + K, bf)
                gather_wait(b)
                out_fire(ph, j, b)
            return carry

        lax.fori_loop(0, (CPP - 2 * K) // D, group, 0)

        for i in range(K):
            j = CPP - K + i
            b = j % D
            out_wait((j + K) % D)
            gather_wait(b)
            out_fire(ph, j, b)
        for i in range(K):
            j = CPP - K + i
            out_wait(j % D)


@jax.jit
def _emb_lookup(idx, emb_weight):
    idxg = idx.reshape(NW, N_CHUNKS, CH).astype(jnp.int32)
    wp = jnp.pad(emb_weight, ((0, 0), (0, 128 - DIM)))
    mesh = plsc.VectorSubcoreMesh(core_axis_name="c", subcore_axis_name="s")
    run = pl.kernel(
        _emb_body,
        out_type=jax.ShapeDtypeStruct((R_TOT, 128), jnp.float32),
        mesh=mesh,
        scratch_types=[
            pltpu.VMEM((CPP, CH), jnp.int32),
            pltpu.VMEM((D, CH, 128), jnp.float32),
        ] + [pltpu.SemaphoreType.DMA] * (2 * D),
        compiler_params=pltpu.CompilerParams(use_tc_tiling_on_sc=False),
    )
    op = run(idxg, wp)
    return op.reshape(B, LP, 128)[:, :L, :DIM]


def kernel(idx, emb_weight):
    return _emb_lookup(idx, emb_weight)
